# baseline (speedup 1.0000x reference)
import jax
import jax.numpy as jnp
from jax.experimental import pallas as pl
from jax.experimental.pallas import tpu as pltpu

_N_EMB = 32
_B = 8
_C = 384
_H = 224
_W = 224
_BCC = 32
_GRID = (_B, _C // _BCC)


def _body(x_ref, tbl_ref, o_ref):
    sums = jnp.sum(x_ref[...], axis=(2, 3))
    mean = sums / float(_H * _W)
    t = jnp.tanh(mean)
    idx = ((t + 1.0) / 2.0 * (_N_EMB - 1)).astype(jnp.int32)
    beta = jnp.zeros((1, _BCC), jnp.float32)
    for e in range(_N_EMB):
        beta = jnp.where(idx == e, tbl_ref[0, e], beta)
    o_ref[...] = beta[None]


def kernel(x, beta_table):
    b, c = x.shape[0], x.shape[1]
    tbl = beta_table.reshape(1, _N_EMB)
    out = pl.pallas_call(
        _body,
        grid=_GRID,
        in_specs=[
            pl.BlockSpec((1, _BCC, _H, _W), lambda i, j: (i, j, 0, 0)),
            pl.BlockSpec(memory_space=pltpu.SMEM),
        ],
        out_specs=pl.BlockSpec(
            (1, 1, _BCC), lambda i, j: (i * (_C // _BCC) + j, 0, 0)
        ),
        out_shape=jax.ShapeDtypeStruct((_B * _C // _BCC, 1, _BCC), jnp.float32),
        compiler_params=pltpu.CompilerParams(
            dimension_semantics=("parallel", "parallel"),
        ),
    )(x, tbl)
    return out.reshape(b, c, 1, 1)
